# baseline (speedup 1.0000x reference)
import functools

import jax
import jax.numpy as jnp
from jax import lax
from jax.experimental import pallas as pl
from jax.experimental.pallas import tpu as pltpu
from jax.experimental.pallas import tpu_sc as plsc

B = 4096
NEG = 4
D = 64
NC = 2
NS = 16
NW = NC * NS
BPW = B // NW
L = 16
GROUPS = BPW // L

_mesh = plsc.VectorSubcoreMesh(core_axis_name="c", subcore_axis_name="s")


@functools.partial(
    pl.kernel,
    mesh=_mesh,
    compiler_params=pltpu.CompilerParams(needs_layout_passes=False,
                                         use_tc_tiling_on_sc=False),
    out_type=jax.ShapeDtypeStruct((B, 1 + NEG), jnp.float32),
    scratch_types=[
        pltpu.VMEM((BPW,), jnp.int32),
        pltpu.VMEM((BPW,), jnp.int32),
        pltpu.VMEM((NEG, BPW), jnp.int32),
        pltpu.VMEM((BPW, D), jnp.float32),
        pltpu.VMEM((BPW, D), jnp.float32),
        pltpu.VMEM((NEG * BPW, D), jnp.float32),
        pltpu.VMEM((BPW, 1 + NEG), jnp.float32),
        pltpu.SemaphoreType.DMA,
    ],
)
def _bpr_sc(user_hbm, pos_hbm, negt_hbm, utab_hbm, itab_hbm, out_hbm,
            uidx, pidx, nidx, urows, prows, nrows, oblk, sem):
    wid = lax.axis_index("s") * NC + lax.axis_index("c")
    base = wid * BPW

    pltpu.sync_copy(user_hbm.at[pl.ds(base, BPW)], uidx)
    pltpu.sync_copy(pos_hbm.at[pl.ds(base, BPW)], pidx)
    for j in range(NEG):
        pltpu.sync_copy(negt_hbm.at[pl.ds(j * B + base, BPW)], nidx.at[j])

    copies = [
        pltpu.async_copy(utab_hbm.at[uidx], urows, sem),
        pltpu.async_copy(itab_hbm.at[pidx], prows, sem),
    ]
    for j in range(NEG):
        copies.append(
            pltpu.async_copy(itab_hbm.at[nidx.at[j]],
                             nrows.at[pl.ds(j * BPW, BPW)], sem))
    for c in copies:
        c.wait()

    iota = lax.iota(jnp.int32, L)
    zero = jnp.zeros((L,), jnp.float32)

    def group_body(g, _):
        r = g * L + iota
        rn = [r + j * BPW for j in range(NEG)]

        def d_body(d, accs):
            dcol = jnp.full((L,), d, jnp.int32)
            uv = plsc.load_gather(urows, [r, dcol])
            pv = plsc.load_gather(prows, [r, dcol])
            nv = [plsc.load_gather(nrows, [rn[j], dcol]) for j in range(NEG)]
            return (accs[0] + uv * pv,) + tuple(
                accs[1 + j] + uv * nv[j] for j in range(NEG))

        accs = lax.fori_loop(0, D, d_body, (zero,) * (1 + NEG))
        for col in range(1 + NEG):
            plsc.store_scatter(oblk, [r, jnp.full((L,), col, jnp.int32)],
                               accs[col])
        return 0

    lax.fori_loop(0, GROUPS, group_body, 0)
    pltpu.sync_copy(oblk, out_hbm.at[pl.ds(base, BPW)])


def kernel(user, pos_item, neg_item, user_table, item_table):
    negt = neg_item.T.reshape(-1)
    return _bpr_sc(user, pos_item, negt, user_table, item_table)
